# Initial kernel scaffold; baseline (speedup 1.0000x reference)
import jax
import jax.numpy as jnp
from jax.experimental import pallas as pl


def kernel(num_attr, cc_attr, y_init, emb_importance, emb_oneway, emb_tunnel, emb_lanes, W, b):
    raise NotImplementedError("write your pallas kernel here")


# speedup vs baseline: 6.1322x; 6.1322x over previous
import functools

import jax
import jax.numpy as jnp
from jax.experimental import pallas as pl

B, L = 4096, 200
N = B * L
BLK = 4096


def _edge_attr_kernel(num_ref, cc_ref, y_ref, ei_ref, eo_ref, et_ref, el_ref,
                      w_ref, b_ref, out_ref):
    cc = cc_ref[...]
    num = num_ref[...]
    y = y_ref[...]

    def onehot(col, width):
        iota = jax.lax.broadcasted_iota(jnp.int32, (1, width), 1)
        return (cc[:, col:col + 1] == iota).astype(jnp.float32)

    a = jnp.concatenate(
        [onehot(0, 8), onehot(1, 2), onehot(2, 2), onehot(3, 6), num, y],
        axis=1)

    w = w_ref[...]
    wf = jnp.concatenate([
        jnp.dot(ei_ref[...], w[0:5], preferred_element_type=jnp.float32),
        jnp.dot(eo_ref[...], w[5:7], preferred_element_type=jnp.float32),
        jnp.dot(et_ref[...], w[7:9], preferred_element_type=jnp.float32),
        jnp.dot(el_ref[...], w[9:12], preferred_element_type=jnp.float32),
        w[12:20],
        w[20:21],
    ], axis=0)

    out = jnp.dot(a, wf, preferred_element_type=jnp.float32) + b_ref[...]
    out_ref[...] = jnp.where(out >= 0, out, 0.01 * out)


@functools.partial(jax.jit, static_argnames=())
def kernel(num_attr, cc_attr, y_init, emb_importance, emb_oneway, emb_tunnel,
           emb_lanes, W, b):
    num2 = num_attr.reshape(N, 8)
    cc2 = cc_attr.reshape(N, 4)
    y2 = y_init.reshape(N, 1)
    b2 = b.reshape(1, 64)

    grid = (N // BLK,)
    out = pl.pallas_call(
        _edge_attr_kernel,
        grid=grid,
        in_specs=[
            pl.BlockSpec((BLK, 8), lambda i: (i, 0)),
            pl.BlockSpec((BLK, 4), lambda i: (i, 0)),
            pl.BlockSpec((BLK, 1), lambda i: (i, 0)),
            pl.BlockSpec(emb_importance.shape, lambda i: (0, 0)),
            pl.BlockSpec(emb_oneway.shape, lambda i: (0, 0)),
            pl.BlockSpec(emb_tunnel.shape, lambda i: (0, 0)),
            pl.BlockSpec(emb_lanes.shape, lambda i: (0, 0)),
            pl.BlockSpec(W.shape, lambda i: (0, 0)),
            pl.BlockSpec((1, 64), lambda i: (0, 0)),
        ],
        out_specs=pl.BlockSpec((BLK, 64), lambda i: (i, 0)),
        out_shape=jax.ShapeDtypeStruct((N, 64), jnp.float32),
    )(num2, cc2, y2, emb_importance, emb_oneway, emb_tunnel, emb_lanes, W, b2)
    return out.reshape(B, L, 64)
